# Initial kernel scaffold; baseline (speedup 1.0000x reference)
import jax
import jax.numpy as jnp
from jax.experimental import pallas as pl


def kernel(x, Adj_, W1, b1, W2, b2):
    raise NotImplementedError("write your pallas kernel here")


# speedup vs baseline: 1.0330x; 1.0330x over previous
import jax
import jax.numpy as jnp
from jax.experimental import pallas as pl
from jax.experimental.pallas import tpu as pltpu

_BI = 400


def _pass1_kernel(adj_ref, x_ref, w1_ref, b1_ref, w2_ref, out_ref):
    adj = adj_ref[...]
    s = jnp.dot(adj, x_ref[...], preferred_element_type=jnp.float32)
    r = jnp.sum(adj, axis=1, keepdims=True)
    h = jnp.maximum(
        jnp.dot(s, w1_ref[...], preferred_element_type=jnp.float32)
        + r * b1_ref[...],
        0.0,
    )
    out_ref[...] = jnp.dot(h, w2_ref[...], preferred_element_type=jnp.float32)


def _pass2_kernel(adj_ref, b_ref, b2_ref, out_ref):
    adj = adj_ref[...]
    o = (
        jnp.dot(adj, b_ref[...], preferred_element_type=jnp.float32)
        + jnp.sum(adj, axis=1, keepdims=True) * b2_ref[...]
    )
    nrm = jnp.sqrt(jnp.sum(o * o, axis=1, keepdims=True))
    out_ref[...] = o / jnp.maximum(nrm, 1e-12)


def kernel(x, Adj_, W1, b1, W2, b2):
    n, in_dim = x.shape
    emb_dim = W2.shape[1]
    b1r = b1.reshape(1, -1)
    b2r = b2.reshape(1, -1)
    grid = (n // _BI,)
    cparams = pltpu.CompilerParams(
        dimension_semantics=("arbitrary",),
    )

    B = pl.pallas_call(
        _pass1_kernel,
        grid=grid,
        in_specs=[
            pl.BlockSpec((_BI, n), lambda i: (i, 0)),
            pl.BlockSpec((n, in_dim), lambda i: (0, 0)),
            pl.BlockSpec(W1.shape, lambda i: (0, 0)),
            pl.BlockSpec(b1r.shape, lambda i: (0, 0)),
            pl.BlockSpec(W2.shape, lambda i: (0, 0)),
        ],
        out_specs=pl.BlockSpec((_BI, emb_dim), lambda i: (i, 0)),
        out_shape=jax.ShapeDtypeStruct((n, emb_dim), jnp.float32),
        compiler_params=cparams,
    )(Adj_, x, W1, b1r, W2)

    emb = pl.pallas_call(
        _pass2_kernel,
        grid=grid,
        in_specs=[
            pl.BlockSpec((_BI, n), lambda i: (i, 0)),
            pl.BlockSpec((n, emb_dim), lambda i: (0, 0)),
            pl.BlockSpec(b2r.shape, lambda i: (0, 0)),
        ],
        out_specs=pl.BlockSpec((_BI, emb_dim), lambda i: (i, 0)),
        out_shape=jax.ShapeDtypeStruct((n, emb_dim), jnp.float32),
        compiler_params=cparams,
    )(Adj_, B, b2r)

    return emb
